# Initial kernel scaffold; baseline (speedup 1.0000x reference)
import jax
import jax.numpy as jnp
from jax.experimental import pallas as pl


def kernel(x, edge_attr, context, params, edge_index):
    raise NotImplementedError("write your pallas kernel here")


# speedup vs baseline: 2.7759x; 2.7759x over previous
import functools

import jax
import jax.numpy as jnp
from jax import lax
from jax.experimental import pallas as pl
from jax.experimental.pallas import tpu as pltpu
from jax.experimental.pallas import tpu_sc as plsc

N = 10000
E = 320000
LATENT = 128

NC = 2
NS = 16
NW = NC * NS
PER_W = E // NW
CHUNK = 80
NCHUNKS = PER_W // CHUNK

_f32 = jnp.float32


def _ln(z, g, b):
    mu = jnp.mean(z, axis=-1, keepdims=True)
    var = jnp.mean((z - mu) ** 2, axis=-1, keepdims=True)
    return (z - mu) * lax.rsqrt(var + 1e-6) * g + b


def _node_encode_body(x_ref, w1, b1, w2, b2, w3, b3, g, be, ws, wd,
                      h_ref, p_ref, q_ref):
    z = jnp.maximum(jnp.dot(x_ref[...], w1[...], preferred_element_type=_f32) + b1[...], 0.0)
    z = jnp.maximum(jnp.dot(z, w2[...], preferred_element_type=_f32) + b2[...], 0.0)
    z = jnp.dot(z, w3[...], preferred_element_type=_f32) + b3[...]
    h = _ln(z, g[...], be[...])
    h_ref[...] = h
    p_ref[...] = jnp.dot(h, ws[...], preferred_element_type=_f32)
    q_ref[...] = jnp.dot(h, wd[...], preferred_element_type=_f32)


def _edge_encode_body(a_ref, w1, b1, w2, b2, w3, b3, g, be, e_ref):
    z = jnp.maximum(jnp.dot(a_ref[...], w1[...], preferred_element_type=_f32) + b1[...], 0.0)
    z = jnp.maximum(jnp.dot(z, w2[...], preferred_element_type=_f32) + b2[...], 0.0)
    z = jnp.dot(z, w3[...], preferred_element_type=_f32) + b3[...]
    e_ref[...] = _ln(z, g[...], be[...])


def _edge_msg_body(g1_ref, g2_ref, e_ref, w1e, b1, w2, b2, w3, b3, g, be, m_ref):
    z = g1_ref[...] + g2_ref[...] + jnp.dot(e_ref[...], w1e[...], preferred_element_type=_f32) + b1[...]
    z = jnp.maximum(z, 0.0)
    z = jnp.maximum(jnp.dot(z, w2[...], preferred_element_type=_f32) + b2[...], 0.0)
    z = jnp.dot(z, w3[...], preferred_element_type=_f32) + b3[...]
    m_ref[...] = _ln(z, g[...], be[...])


def _node_update_body(h_ref, pa_ref, pb_ref, wh, wp, b1, w2, b2, w3, b3, g, be,
                      ws, wd, h_out, p_ref, q_ref):
    h = h_ref[...]
    pooled = pa_ref[...] + pb_ref[...]
    z = (jnp.dot(h, wh[...], preferred_element_type=_f32)
         + jnp.dot(pooled, wp[...], preferred_element_type=_f32) + b1[...])
    z = jnp.maximum(z, 0.0)
    z = jnp.maximum(jnp.dot(z, w2[...], preferred_element_type=_f32) + b2[...], 0.0)
    z = jnp.dot(z, w3[...], preferred_element_type=_f32) + b3[...]
    h = h + _ln(z, g[...], be[...])
    h_out[...] = h
    p_ref[...] = jnp.dot(h, ws[...], preferred_element_type=_f32)
    q_ref[...] = jnp.dot(h, wd[...], preferred_element_type=_f32)


def _decoder_body(h_ref, w1, b1, w2, b2, w3, b3, o_ref):
    z = jnp.maximum(jnp.dot(h_ref[...], w1[...], preferred_element_type=_f32) + b1[...], 0.0)
    z = jnp.maximum(jnp.dot(z, w2[...], preferred_element_type=_f32) + b2[...], 0.0)
    o_ref[...] = jnp.dot(z, w3[...], preferred_element_type=_f32) + b3[...]


def _row_spec(block_rows, cols):
    return pl.BlockSpec((block_rows, cols), lambda i: (i, 0))


def _full_spec(shape):
    return pl.BlockSpec(shape, lambda i: tuple(0 for _ in shape))


def _tc_call(body, grid, in_arrays, blocked_cols, block_rows, out_shapes):
    in_specs = []
    for a, bc in in_arrays:
        if bc is None:
            in_specs.append(_full_spec(a.shape))
        else:
            in_specs.append(_row_spec(block_rows, bc))
    out_specs = tuple(_row_spec(block_rows, s[1]) for s in out_shapes)
    out_shape = tuple(jax.ShapeDtypeStruct(s, _f32) for s in out_shapes)
    if len(out_shapes) == 1:
        out_specs = out_specs[0]
        out_shape = out_shape[0]
    return pl.pallas_call(
        body, grid=(grid,),
        in_specs=in_specs, out_specs=out_specs, out_shape=out_shape,
    )(*[a for a, _ in in_arrays])


_MESH = plsc.VectorSubcoreMesh(core_axis_name="c", subcore_axis_name="s")


@functools.partial(
    pl.kernel,
    out_type=(jax.ShapeDtypeStruct((E, LATENT), _f32),
              jax.ShapeDtypeStruct((E, LATENT), _f32)),
    mesh=_MESH,
    scratch_types=[
        pltpu.VMEM((CHUNK,), jnp.int32),
        pltpu.VMEM((CHUNK,), jnp.int32),
        pltpu.VMEM((CHUNK, LATENT), _f32),
        pltpu.VMEM((CHUNK, LATENT), _f32),
        pltpu.SemaphoreType.DMA,
        pltpu.SemaphoreType.DMA,
    ],
)
def _sc_gather(p_hbm, q_hbm, src_hbm, dst_hbm, g1_hbm, g2_hbm,
               si, di, r1, r2, sem1, sem2):
    wid = lax.axis_index("s") * NC + lax.axis_index("c")
    base = wid * PER_W

    def body(ci, carry):
        off = base + ci * CHUNK
        pltpu.sync_copy(src_hbm.at[pl.ds(off, CHUNK)], si)
        pltpu.sync_copy(dst_hbm.at[pl.ds(off, CHUNK)], di)
        c1 = pltpu.async_copy(p_hbm.at[si], r1, sem1)
        c2 = pltpu.async_copy(q_hbm.at[di], r2, sem2)
        c1.wait()
        c2.wait()
        pltpu.sync_copy(r1, g1_hbm.at[pl.ds(off, CHUNK)])
        pltpu.sync_copy(r2, g2_hbm.at[pl.ds(off, CHUNK)])
        return carry

    lax.fori_loop(0, NCHUNKS, body, 0)


N_PAD = 10240
_ROWS_PER_TILE = N_PAD // NS


@functools.partial(
    pl.kernel,
    out_type=jax.ShapeDtypeStruct((NC, N_PAD, LATENT), _f32),
    mesh=_MESH,
    scratch_types=[
        pltpu.VMEM((CHUNK,), jnp.int32),
        pltpu.VMEM((CHUNK, LATENT), _f32),
        pltpu.VMEM_SHARED((N_PAD, LATENT), _f32),
    ],
)
def _sc_scatter(msg_hbm, dst_hbm, zeros_hbm, out_hbm, di, rows, acc):
    cid = lax.axis_index("c")
    sid = lax.axis_index("s")
    wid = sid * NC + cid
    base = wid * PER_W
    r0 = sid * _ROWS_PER_TILE

    pltpu.sync_copy(zeros_hbm.at[pl.ds(r0, _ROWS_PER_TILE)],
                    acc.at[pl.ds(r0, _ROWS_PER_TILE)])
    plsc.subcore_barrier()

    def body(ci, carry):
        off = base + ci * CHUNK
        pltpu.sync_copy(dst_hbm.at[pl.ds(off, CHUNK)], di)
        pltpu.sync_copy(msg_hbm.at[pl.ds(off, CHUNK)], rows)
        pltpu.sync_copy(rows, acc.at[di], add=True)
        return carry

    lax.fori_loop(0, NCHUNKS, body, 0)
    plsc.subcore_barrier()
    pltpu.sync_copy(acc.at[pl.ds(r0, _ROWS_PER_TILE)],
                    out_hbm.at[cid, pl.ds(r0, _ROWS_PER_TILE)])


def _b2(v):
    return v.reshape(1, -1)


def kernel(x, edge_attr, context, params, edge_index):
    del context
    src = edge_index[0].astype(jnp.int32)
    dst = edge_index[1].astype(jnp.int32)

    ne = params['node_enc']
    ee = params['edge_enc']
    dec = params['decoder']
    procs = params['processors']

    w1s = [p['msg']['layers'][0][0][:LATENT] for p in procs]
    w1d = [p['msg']['layers'][0][0][LATENT:2 * LATENT] for p in procs]
    w1e = [p['msg']['layers'][0][0][2 * LATENT:] for p in procs]

    BN = 2000
    BE = 2000

    (w, b), (w2, b2), (w3, b3) = ne['layers']
    g, be_ = ne['ln']
    h, P, Q = _tc_call(
        _node_encode_body, N // BN,
        [(x, LATENT), (w, None), (_b2(b), None), (w2, None), (_b2(b2), None),
         (w3, None), (_b2(b3), None), (_b2(g), None), (_b2(be_), None),
         (w1s[0], None), (w1d[0], None)],
        None, BN, [(N, LATENT), (N, LATENT), (N, LATENT)])

    (w, b), (w2, b2), (w3, b3) = ee['layers']
    g, be_ = ee['ln']
    e = _tc_call(
        _edge_encode_body, E // BE,
        [(edge_attr, edge_attr.shape[1]), (w, None), (_b2(b), None),
         (w2, None), (_b2(b2), None), (w3, None), (_b2(b3), None),
         (_b2(g), None), (_b2(be_), None)],
        None, BE, [(E, LATENT)])

    zeros_n = jnp.zeros((N_PAD, LATENT), _f32)

    for i, p in enumerate(procs):
        G1, G2 = _sc_gather(P, Q, src, dst)

        (_, b1m), (w2m, b2m), (w3m, b3m) = p['msg']['layers']
        gm, bem = p['msg']['ln']
        msg = _tc_call(
            _edge_msg_body, E // BE,
            [(G1, LATENT), (G2, LATENT), (e, LATENT), (w1e[i], None),
             (_b2(b1m), None), (w2m, None), (_b2(b2m), None), (w3m, None),
             (_b2(b3m), None), (_b2(gm), None), (_b2(bem), None)],
            None, BE, [(E, LATENT)])

        pooled2 = _sc_scatter(msg, dst, zeros_n)

        (w1u, b1u), (w2u, b2u), (w3u, b3u) = p['upd']['layers']
        gu, beu = p['upd']['ln']
        wh, wp = w1u[:LATENT], w1u[LATENT:]
        nxt = i + 1 if i + 1 < len(procs) else i
        h, P, Q = _tc_call(
            _node_update_body, N // BN,
            [(h, LATENT), (pooled2[0][:N], LATENT), (pooled2[1][:N], LATENT),
             (wh, None), (wp, None), (_b2(b1u), None), (w2u, None),
             (_b2(b2u), None), (w3u, None), (_b2(b3u), None), (_b2(gu), None),
             (_b2(beu), None), (w1s[nxt], None), (w1d[nxt], None)],
            None, BN, [(N, LATENT), (N, LATENT), (N, LATENT)])

    (w, b), (w2, b2), (w3, b3) = dec['layers']
    out_dim = w3.shape[1]
    w3p = jnp.zeros((LATENT, LATENT), _f32).at[:, :out_dim].set(w3)
    b3p = jnp.zeros((LATENT,), _f32).at[:out_dim].set(b3)
    out = _tc_call(
        _decoder_body, N // BN,
        [(h, LATENT), (w, None), (_b2(b), None), (w2, None), (_b2(b2), None),
         (w3p, None), (_b2(b3p), None)],
        None, BN, [(N, LATENT)])
    return out[:, :out_dim]


# speedup vs baseline: 4.1761x; 1.5044x over previous
import functools

import jax
import jax.numpy as jnp
from jax import lax
from jax.experimental import pallas as pl
from jax.experimental.pallas import tpu as pltpu
from jax.experimental.pallas import tpu_sc as plsc

N = 10000
E = 320000
LATENT = 128

NC = 2
NS = 16
NW = NC * NS
PER_W = E // NW
CHUNK = 80
NCHUNKS = PER_W // CHUNK

_f32 = jnp.float32


def _ln(z, g, b):
    mu = jnp.mean(z, axis=-1, keepdims=True)
    var = jnp.mean((z - mu) ** 2, axis=-1, keepdims=True)
    return (z - mu) * lax.rsqrt(var + 1e-6) * g + b


def _node_encode_body(x_ref, w1, b1, w2, b2, w3, b3, g, be, ws, wd,
                      h_ref, p_ref, q_ref):
    z = jnp.maximum(jnp.dot(x_ref[...], w1[...], preferred_element_type=_f32) + b1[...], 0.0)
    z = jnp.maximum(jnp.dot(z, w2[...], preferred_element_type=_f32) + b2[...], 0.0)
    z = jnp.dot(z, w3[...], preferred_element_type=_f32) + b3[...]
    h = _ln(z, g[...], be[...])
    h_ref[...] = h
    p_ref[...] = jnp.dot(h, ws[...], preferred_element_type=_f32)
    q_ref[...] = jnp.dot(h, wd[...], preferred_element_type=_f32)


def _edge_encode_body(a_ref, w1, b1, w2, b2, w3, b3, g, be, e_ref):
    z = jnp.maximum(jnp.dot(a_ref[...], w1[...], preferred_element_type=_f32) + b1[...], 0.0)
    z = jnp.maximum(jnp.dot(z, w2[...], preferred_element_type=_f32) + b2[...], 0.0)
    z = jnp.dot(z, w3[...], preferred_element_type=_f32) + b3[...]
    e_ref[...] = _ln(z, g[...], be[...])


def _edge_msg_body(g1_ref, g2_ref, e_ref, w1e, b1, w2, b2, w3, b3, g, be, m_ref):
    z = g1_ref[...] + g2_ref[...] + jnp.dot(e_ref[...], w1e[...], preferred_element_type=_f32) + b1[...]
    z = jnp.maximum(z, 0.0)
    z = jnp.maximum(jnp.dot(z, w2[...], preferred_element_type=_f32) + b2[...], 0.0)
    z = jnp.dot(z, w3[...], preferred_element_type=_f32) + b3[...]
    m_ref[...] = _ln(z, g[...], be[...])


def _node_update_body(h_ref, pa_ref, pb_ref, wh, wp, b1, w2, b2, w3, b3, g, be,
                      ws, wd, h_out, p_ref, q_ref):
    h = h_ref[...]
    pooled = pa_ref[...] + pb_ref[...]
    z = (jnp.dot(h, wh[...], preferred_element_type=_f32)
         + jnp.dot(pooled, wp[...], preferred_element_type=_f32) + b1[...])
    z = jnp.maximum(z, 0.0)
    z = jnp.maximum(jnp.dot(z, w2[...], preferred_element_type=_f32) + b2[...], 0.0)
    z = jnp.dot(z, w3[...], preferred_element_type=_f32) + b3[...]
    h = h + _ln(z, g[...], be[...])
    h_out[...] = h
    p_ref[...] = jnp.dot(h, ws[...], preferred_element_type=_f32)
    q_ref[...] = jnp.dot(h, wd[...], preferred_element_type=_f32)


def _decoder_body(h_ref, w1, b1, w2, b2, w3, b3, o_ref):
    z = jnp.maximum(jnp.dot(h_ref[...], w1[...], preferred_element_type=_f32) + b1[...], 0.0)
    z = jnp.maximum(jnp.dot(z, w2[...], preferred_element_type=_f32) + b2[...], 0.0)
    o_ref[...] = jnp.dot(z, w3[...], preferred_element_type=_f32) + b3[...]


def _row_spec(block_rows, cols):
    return pl.BlockSpec((block_rows, cols), lambda i: (i, 0))


def _full_spec(shape):
    return pl.BlockSpec(shape, lambda i: tuple(0 for _ in shape))


def _tc_call(body, grid, in_arrays, blocked_cols, block_rows, out_shapes):
    in_specs = []
    for a, bc in in_arrays:
        if bc is None:
            in_specs.append(_full_spec(a.shape))
        else:
            in_specs.append(_row_spec(block_rows, bc))
    out_specs = tuple(_row_spec(block_rows, s[1]) for s in out_shapes)
    out_shape = tuple(jax.ShapeDtypeStruct(s, _f32) for s in out_shapes)
    if len(out_shapes) == 1:
        out_specs = out_specs[0]
        out_shape = out_shape[0]
    return pl.pallas_call(
        body, grid=(grid,),
        in_specs=in_specs, out_specs=out_specs, out_shape=out_shape,
    )(*[a for a, _ in in_arrays])


_MESH = plsc.VectorSubcoreMesh(core_axis_name="c", subcore_axis_name="s")

NBUF = 5
NGROUPS = NCHUNKS // NBUF
CHUNK_S = 40
NCHUNKS_S = PER_W // CHUNK_S
NGROUPS_S = NCHUNKS_S // NBUF


@functools.partial(
    pl.kernel,
    out_type=(jax.ShapeDtypeStruct((E, LATENT), _f32),
              jax.ShapeDtypeStruct((E, LATENT), _f32)),
    mesh=_MESH,
    scratch_types=(
        [pltpu.VMEM((CHUNK,), jnp.int32) for _ in range(NBUF)]
        + [pltpu.VMEM((CHUNK,), jnp.int32) for _ in range(NBUF)]
        + [pltpu.VMEM((CHUNK, LATENT), _f32) for _ in range(NBUF)]
        + [pltpu.VMEM((CHUNK, LATENT), _f32) for _ in range(NBUF)]
        + [pltpu.SemaphoreType.DMA for _ in range(3 * NBUF)]
    ),
)
def _sc_gather(p_hbm, q_hbm, src_hbm, dst_hbm, g1_hbm, g2_hbm, *scratch):
    si = scratch[:NBUF]
    di = scratch[NBUF:2 * NBUF]
    r1 = scratch[2 * NBUF:3 * NBUF]
    r2 = scratch[3 * NBUF:4 * NBUF]
    sems = scratch[4 * NBUF:]
    sem_i = sems[:NBUF]
    sem_g = sems[NBUF:2 * NBUF]
    sem_o = sems[2 * NBUF:]

    wid = lax.axis_index("s") * NC + lax.axis_index("c")
    base = wid * PER_W

    for b in range(NBUF):
        off = base + b * CHUNK
        pltpu.async_copy(src_hbm.at[pl.ds(off, CHUNK)], si[b], sem_i[b])
        pltpu.async_copy(dst_hbm.at[pl.ds(off, CHUNK)], di[b], sem_i[b])

    @pl.loop(0, NGROUPS)
    def _group(g):
        for b in range(NBUF):
            ci = g * NBUF + b
            off = base + ci * CHUNK
            prev_off = off - NBUF * CHUNK

            @pl.when(g > 0)
            def _():
                pltpu.make_async_copy(
                    r1[b], g1_hbm.at[pl.ds(prev_off, CHUNK)], sem_o[b]).wait()
                pltpu.make_async_copy(
                    r2[b], g2_hbm.at[pl.ds(prev_off, CHUNK)], sem_o[b]).wait()

            pltpu.make_async_copy(src_hbm.at[pl.ds(off, CHUNK)], si[b], sem_i[b]).wait()
            pltpu.make_async_copy(dst_hbm.at[pl.ds(off, CHUNK)], di[b], sem_i[b]).wait()
            pltpu.async_copy(p_hbm.at[si[b]], r1[b], sem_g[b])
            pltpu.async_copy(q_hbm.at[di[b]], r2[b], sem_g[b])

        for b in range(NBUF):
            ci = g * NBUF + b
            off = base + ci * CHUNK
            nxt_off = off + NBUF * CHUNK
            pltpu.make_async_copy(p_hbm.at[si[b]], r1[b], sem_g[b]).wait()
            pltpu.make_async_copy(q_hbm.at[di[b]], r2[b], sem_g[b]).wait()
            pltpu.async_copy(r1[b], g1_hbm.at[pl.ds(off, CHUNK)], sem_o[b])
            pltpu.async_copy(r2[b], g2_hbm.at[pl.ds(off, CHUNK)], sem_o[b])

            @pl.when(g < NGROUPS - 1)
            def _():
                pltpu.async_copy(src_hbm.at[pl.ds(nxt_off, CHUNK)], si[b], sem_i[b])
                pltpu.async_copy(dst_hbm.at[pl.ds(nxt_off, CHUNK)], di[b], sem_i[b])

    last = base + (NGROUPS - 1) * NBUF * CHUNK
    for b in range(NBUF):
        off = last + b * CHUNK
        pltpu.make_async_copy(r1[b], g1_hbm.at[pl.ds(off, CHUNK)], sem_o[b]).wait()
        pltpu.make_async_copy(r2[b], g2_hbm.at[pl.ds(off, CHUNK)], sem_o[b]).wait()


N_PAD = 10240
_ROWS_PER_TILE = N_PAD // NS


@functools.partial(
    pl.kernel,
    out_type=jax.ShapeDtypeStruct((NC, N_PAD, LATENT), _f32),
    mesh=_MESH,
    scratch_types=(
        [pltpu.VMEM((CHUNK_S,), jnp.int32) for _ in range(NBUF)]
        + [pltpu.VMEM((CHUNK_S, LATENT), _f32) for _ in range(NBUF)]
        + [pltpu.VMEM_SHARED((N_PAD, LATENT), _f32)]
        + [pltpu.SemaphoreType.DMA for _ in range(NBUF)]
    ),
)
def _sc_scatter(msg_hbm, dst_hbm, zeros_hbm, out_hbm, *scratch):
    di = scratch[:NBUF]
    rows = scratch[NBUF:2 * NBUF]
    acc = scratch[2 * NBUF]
    sem_l = scratch[2 * NBUF + 1:]

    cid = lax.axis_index("c")
    sid = lax.axis_index("s")
    wid = sid * NC + cid
    base = wid * PER_W
    r0 = sid * _ROWS_PER_TILE

    pltpu.sync_copy(zeros_hbm.at[pl.ds(r0, _ROWS_PER_TILE)],
                    acc.at[pl.ds(r0, _ROWS_PER_TILE)])
    plsc.subcore_barrier()

    for b in range(NBUF):
        off = base + b * CHUNK_S
        pltpu.async_copy(dst_hbm.at[pl.ds(off, CHUNK_S)], di[b], sem_l[b])
        pltpu.async_copy(msg_hbm.at[pl.ds(off, CHUNK_S)], rows[b], sem_l[b])

    @pl.loop(0, NGROUPS_S)
    def _group(g):
        for b in range(NBUF):
            ci = g * NBUF + b
            off = base + ci * CHUNK_S
            nxt_off = off + NBUF * CHUNK_S
            pltpu.make_async_copy(dst_hbm.at[pl.ds(off, CHUNK_S)], di[b], sem_l[b]).wait()
            pltpu.make_async_copy(msg_hbm.at[pl.ds(off, CHUNK_S)], rows[b], sem_l[b]).wait()
            pltpu.sync_copy(rows[b], acc.at[di[b]], add=True)

            @pl.when(g < NGROUPS_S - 1)
            def _():
                pltpu.async_copy(dst_hbm.at[pl.ds(nxt_off, CHUNK_S)], di[b], sem_l[b])
                pltpu.async_copy(msg_hbm.at[pl.ds(nxt_off, CHUNK_S)], rows[b], sem_l[b])

    plsc.subcore_barrier()
    pltpu.sync_copy(acc.at[pl.ds(r0, _ROWS_PER_TILE)],
                    out_hbm.at[cid, pl.ds(r0, _ROWS_PER_TILE)])


def _b2(v):
    return v.reshape(1, -1)


def kernel(x, edge_attr, context, params, edge_index):
    del context
    src = edge_index[0].astype(jnp.int32)
    dst = edge_index[1].astype(jnp.int32)

    ne = params['node_enc']
    ee = params['edge_enc']
    dec = params['decoder']
    procs = params['processors']

    w1s = [p['msg']['layers'][0][0][:LATENT] for p in procs]
    w1d = [p['msg']['layers'][0][0][LATENT:2 * LATENT] for p in procs]
    w1e = [p['msg']['layers'][0][0][2 * LATENT:] for p in procs]

    BN = 2000
    BE = 2000

    (w, b), (w2, b2), (w3, b3) = ne['layers']
    g, be_ = ne['ln']
    h, P, Q = _tc_call(
        _node_encode_body, N // BN,
        [(x, LATENT), (w, None), (_b2(b), None), (w2, None), (_b2(b2), None),
         (w3, None), (_b2(b3), None), (_b2(g), None), (_b2(be_), None),
         (w1s[0], None), (w1d[0], None)],
        None, BN, [(N, LATENT), (N, LATENT), (N, LATENT)])

    (w, b), (w2, b2), (w3, b3) = ee['layers']
    g, be_ = ee['ln']
    e = _tc_call(
        _edge_encode_body, E // BE,
        [(edge_attr, edge_attr.shape[1]), (w, None), (_b2(b), None),
         (w2, None), (_b2(b2), None), (w3, None), (_b2(b3), None),
         (_b2(g), None), (_b2(be_), None)],
        None, BE, [(E, LATENT)])

    zeros_n = jnp.zeros((N_PAD, LATENT), _f32)

    for i, p in enumerate(procs):
        G1, G2 = _sc_gather(P, Q, src, dst)

        (_, b1m), (w2m, b2m), (w3m, b3m) = p['msg']['layers']
        gm, bem = p['msg']['ln']
        msg = _tc_call(
            _edge_msg_body, E // BE,
            [(G1, LATENT), (G2, LATENT), (e, LATENT), (w1e[i], None),
             (_b2(b1m), None), (w2m, None), (_b2(b2m), None), (w3m, None),
             (_b2(b3m), None), (_b2(gm), None), (_b2(bem), None)],
            None, BE, [(E, LATENT)])

        pooled2 = _sc_scatter(msg, dst, zeros_n)

        (w1u, b1u), (w2u, b2u), (w3u, b3u) = p['upd']['layers']
        gu, beu = p['upd']['ln']
        wh, wp = w1u[:LATENT], w1u[LATENT:]
        nxt = i + 1 if i + 1 < len(procs) else i
        h, P, Q = _tc_call(
            _node_update_body, N // BN,
            [(h, LATENT), (pooled2[0][:N], LATENT), (pooled2[1][:N], LATENT),
             (wh, None), (wp, None), (_b2(b1u), None), (w2u, None),
             (_b2(b2u), None), (w3u, None), (_b2(b3u), None), (_b2(gu), None),
             (_b2(beu), None), (w1s[nxt], None), (w1d[nxt], None)],
            None, BN, [(N, LATENT), (N, LATENT), (N, LATENT)])

    (w, b), (w2, b2), (w3, b3) = dec['layers']
    out_dim = w3.shape[1]
    w3p = jnp.zeros((LATENT, LATENT), _f32).at[:, :out_dim].set(w3)
    b3p = jnp.zeros((LATENT,), _f32).at[:out_dim].set(b3)
    out = _tc_call(
        _decoder_body, N // BN,
        [(h, LATENT), (w, None), (_b2(b), None), (w2, None), (_b2(b2), None),
         (w3p, None), (_b2(b3p), None)],
        None, BN, [(N, LATENT)])
    return out[:, :out_dim]
